# baseline (speedup 1.0000x reference)
import functools

import jax
import jax.numpy as jnp
from jax import lax
from jax.experimental import pallas as pl
from jax.experimental.pallas import tpu as pltpu
from jax.experimental.pallas import tpu_sc as plsc

N_ATOMS = 100000
EMB = 128
TABLE_ROWS = 101
CHUNK = 128

_info = plsc.get_sparse_core_info()
NC = _info.num_cores
NS = _info.num_subcores
NW = NC * NS

CHUNKS_PER_W = -(-N_ATOMS // (CHUNK * NW))
SPAN = CHUNKS_PER_W * CHUNK
NBUF = 3
AHEAD = 2


def _make_lookup():
    mesh = plsc.VectorSubcoreMesh(core_axis_name="c", subcore_axis_name="s")

    @functools.partial(
        pl.kernel,
        mesh=mesh,
        compiler_params=pltpu.CompilerParams(needs_layout_passes=False),
        out_type=jax.ShapeDtypeStruct((N_ATOMS, EMB), jnp.float32),
        scratch_types=[
            pltpu.VMEM((SPAN,), jnp.int32),
            [pltpu.VMEM((CHUNK, EMB), jnp.float32) for _ in range(NBUF)],
            pltpu.VMEM_SHARED((TABLE_ROWS, EMB), jnp.float32),
            [pltpu.SemaphoreType.DMA for _ in range(NBUF)],
            [pltpu.SemaphoreType.DMA for _ in range(NBUF)],
        ],
    )
    def lookup(z_hbm, table_hbm_pad, out_hbm, idx_all, rows, table_sh,
               gsem, wsem):
        sid = lax.axis_index("s")
        wid = sid * NC + lax.axis_index("c")

        @pl.when(sid == 0)
        def _():
            pltpu.sync_copy(table_hbm_pad, table_sh)

        start = jnp.minimum(wid * SPAN, N_ATOMS - SPAN)
        pltpu.sync_copy(z_hbm.at[pl.ds(start, SPAN)], idx_all)

        plsc.subcore_barrier()

        def issue_gather(k, b):
            src = table_hbm_pad if k % 4 == 3 else table_sh
            pltpu.async_copy(
                src.at[idx_all.at[pl.ds(k * CHUNK, CHUNK)]],
                rows[b], gsem[b])

        def drain(sem, buf):
            pltpu.make_async_copy(out_hbm.at[pl.ds(0, CHUNK)], buf,
                                  sem).wait()

        for k in range(AHEAD):
            issue_gather(k, k % NBUF)

        for e in range(CHUNKS_PER_W):
            if e + AHEAD < CHUNKS_PER_W:
                b1 = (e + AHEAD) % NBUF
                if e + AHEAD >= NBUF:
                    drain(wsem[b1], rows[b1])
                issue_gather(e + AHEAD, b1)
            b = e % NBUF
            drain(gsem[b], rows[b])
            pltpu.async_copy(rows[b],
                             out_hbm.at[pl.ds(start + e * CHUNK, CHUNK)],
                             wsem[b])

        for e in range(CHUNKS_PER_W - NBUF, CHUNKS_PER_W):
            drain(wsem[e % NBUF], rows[e % NBUF])

    return lookup


_lookup = _make_lookup()


def kernel(Z, W):
    W_pad = jnp.concatenate([jnp.zeros((1, EMB), jnp.float32), W], axis=0)
    return _lookup(Z, W_pad)


# speedup vs baseline: 1.6830x; 1.6830x over previous
import functools

import jax
import jax.numpy as jnp
from jax import lax
from jax.experimental import pallas as pl
from jax.experimental.pallas import tpu as pltpu
from jax.experimental.pallas import tpu_sc as plsc

N_ATOMS = 100000
EMB = 128
TABLE_ROWS = 101
CHUNK = 128

_info = plsc.get_sparse_core_info()
NC = _info.num_cores
NS = _info.num_subcores
NW = NC * NS

CHUNKS_PER_W = -(-N_ATOMS // (CHUNK * NW))
SPAN = CHUNKS_PER_W * CHUNK
PAIRS = CHUNKS_PER_W // 2


def _make_lookup():
    mesh = plsc.VectorSubcoreMesh(core_axis_name="c", subcore_axis_name="s")

    @functools.partial(
        pl.kernel,
        mesh=mesh,
        out_type=jax.ShapeDtypeStruct((N_ATOMS, EMB), jnp.float32),
        scratch_types=[
            pltpu.VMEM((SPAN,), jnp.int32),
            pltpu.VMEM((CHUNK, EMB), jnp.float32),
            pltpu.VMEM((CHUNK, EMB), jnp.float32),
            pltpu.VMEM_SHARED((TABLE_ROWS, EMB), jnp.float32),
            pltpu.SemaphoreType.DMA,
            pltpu.SemaphoreType.DMA,
            pltpu.SemaphoreType.DMA,
            pltpu.SemaphoreType.DMA,
        ],
    )
    def lookup(z_hbm, table_hbm, out_hbm, idx_all, rows0, rows1,
               table_sh, gsem0, gsem1, wsem0, wsem1):
        sid = lax.axis_index("s")
        wid = sid * NC + lax.axis_index("c")

        @pl.when(sid == 0)
        def _():
            pltpu.sync_copy(table_hbm, table_sh.at[pl.ds(1, TABLE_ROWS - 1)])

        start = jnp.minimum(wid * SPAN, N_ATOMS - SPAN)
        pltpu.sync_copy(z_hbm.at[pl.ds(start, SPAN)], idx_all)

        plsc.subcore_barrier()

        rows = (rows0, rows1)
        gsem = (gsem0, gsem1)
        wsem = (wsem0, wsem1)

        def issue_gather(k, b):
            pltpu.async_copy(
                table_sh.at[idx_all.at[pl.ds(k * CHUNK, CHUNK)]],
                rows[b], gsem[b])

        def drain(sem, b):
            pltpu.make_async_copy(out_hbm.at[pl.ds(0, CHUNK)], rows[b],
                                  sem).wait()

        issue_gather(0, 0)

        def pair_body(p, carry):
            for b in range(2):
                k = 2 * p + b
                if b == 0:
                    @pl.when(p > 0)
                    def _():
                        drain(wsem[1], 1)
                else:
                    drain(wsem[0], 0)
                issue_gather(k + 1, 1 - b)
                drain(gsem[b], b)
                pltpu.async_copy(rows[b],
                                 out_hbm.at[pl.ds(start + k * CHUNK, CHUNK)],
                                 wsem[b])
            return carry

        lax.fori_loop(0, PAIRS, pair_body, 0)

        k_last = 2 * PAIRS
        drain(wsem[1], 1)
        drain(gsem[0], 0)
        pltpu.sync_copy(rows0, out_hbm.at[pl.ds(start + k_last * CHUNK, CHUNK)])

    return lookup


_lookup = _make_lookup()


def kernel(Z, W):
    return _lookup(Z, W)
